# baseline (device time: 56717 ns/iter reference)
import functools

import jax
import jax.numpy as jnp
from jax import lax
from jax.experimental import pallas as pl
from jax.experimental.pallas import tpu as pltpu

B = 8
H = 8
D = 64
BS = 16
NB = 64
NPAGES_LOCAL = 64
HD = H * D
ROWS = NPAGES_LOCAL * BS
SCALE = D ** -0.5


def _body(k_ref, v_ref, qmat_ref, bt_ref, lens_ref, out_ref,
          kf, vf, ks, vs, send_sems, recv_sems):
    my_x = lax.axis_index("x")
    my_y = lax.axis_index("y")
    my_z = lax.axis_index("z")
    nbr = (1 - my_x, my_y, my_z)

    kf[pl.ds(0, ROWS), :] = k_ref[:, :].astype(jnp.bfloat16)
    vf[pl.ds(0, ROWS), :] = v_ref[:, :].astype(jnp.bfloat16)

    barrier_sem = pltpu.get_barrier_semaphore()
    pl.semaphore_signal(barrier_sem, inc=1, device_id=nbr,
                        device_id_type=pl.DeviceIdType.MESH)
    pl.semaphore_wait(barrier_sem, 1)

    rk = pltpu.make_async_remote_copy(
        src_ref=kf.at[pl.ds(0, ROWS), :],
        dst_ref=kf.at[pl.ds(ROWS, ROWS), :],
        send_sem=send_sems.at[0],
        recv_sem=recv_sems.at[0],
        device_id=nbr,
        device_id_type=pl.DeviceIdType.MESH,
    )
    rv = pltpu.make_async_remote_copy(
        src_ref=vf.at[pl.ds(0, ROWS), :],
        dst_ref=vf.at[pl.ds(ROWS, ROWS), :],
        send_sem=send_sems.at[1],
        recv_sem=recv_sems.at[1],
        device_id=nbr,
        device_id_type=pl.DeviceIdType.MESH,
    )
    rk.start()
    rv.start()

    def gather_pass(want_local):
        def per_batch_gather(i, carry):
            def gather_one(j, c):
                p = bt_ref[i, j]
                is_local = (p // NPAGES_LOCAL) == my_x

                @pl.when(is_local == want_local)
                def _():
                    off = (jnp.where(is_local, 0, ROWS)
                           + (p % NPAGES_LOCAL) * BS)
                    dst = i * ROWS + j * BS
                    ks[pl.ds(dst, BS), :] = kf[pl.ds(off, BS), :]
                    vs[pl.ds(dst, BS), :] = vf[pl.ds(off, BS), :]
                return c
            return lax.fori_loop(0, NB, gather_one, carry)
        lax.fori_loop(0, B, per_batch_gather, 0)

    gather_pass(True)
    rk.wait()
    rv.wait()
    gather_pass(False)

    kmask = lax.broadcasted_iota(jnp.int32, (NB * BS, 1), 0)

    def per_batch(i, carry):
        n_valid = lens_ref[i] * BS
        ksb = ks[pl.ds(i * ROWS, ROWS), :]
        vsb = vs[pl.ds(i * ROWS, ROWS), :]
        qmat = qmat_ref[pl.ds(i * HD, HD), :]
        s = lax.dot_general(
            ksb, qmat, (((1,), (0,)), ((), ())),
            preferred_element_type=jnp.float32,
        ) * SCALE
        s = jnp.where(kmask < n_valid, s, -1e30)
        m = jnp.max(s, axis=0, keepdims=True)
        p_ = jnp.exp(s - m)
        denom = jnp.sum(p_, axis=0, keepdims=True)
        r = lax.dot_general(
            p_.astype(jnp.bfloat16), vsb, (((0,), (0,)), ((), ())),
            preferred_element_type=jnp.float32,
        )
        outs = [
            r[h:h + 1, h * D:(h + 1) * D] / denom[0:1, h:h + 1]
            for h in range(H)
        ]
        out_ref[pl.ds(i, 1), :] = jnp.concatenate(outs, axis=1)
        return carry

    lax.fori_loop(0, B, per_batch, 0)

    @functools.partial(pl.run_scoped, exit_sem=pltpu.SemaphoreType.REGULAR)
    def _(exit_sem):
        pl.semaphore_signal(exit_sem, inc=1, device_id=nbr,
                            device_id_type=pl.DeviceIdType.MESH)
        pl.semaphore_wait(exit_sem, 1)


def kernel(Q, K, V, bt, lens):
    q2 = Q.reshape(B, HD)
    k2 = K.reshape(ROWS, HD)
    v2 = V.reshape(ROWS, HD)

    head_of_col = jnp.arange(HD, dtype=jnp.int32) // D
    head_mask = head_of_col[:, None] == jnp.arange(H, dtype=jnp.int32)[None]
    qmat = (q2[:, :, None] * head_mask[None]).astype(jnp.bfloat16)
    qmat = qmat.reshape(B * HD, H)

    out2 = pl.pallas_call(
        _body,
        out_shape=jax.ShapeDtypeStruct((B, HD), jnp.float32),
        in_specs=[
            pl.BlockSpec(memory_space=pltpu.VMEM),
            pl.BlockSpec(memory_space=pltpu.VMEM),
            pl.BlockSpec(memory_space=pltpu.VMEM),
            pl.BlockSpec(memory_space=pltpu.SMEM),
            pl.BlockSpec(memory_space=pltpu.SMEM),
        ],
        out_specs=pl.BlockSpec(memory_space=pltpu.VMEM),
        scratch_shapes=[
            pltpu.VMEM((2 * ROWS, HD), jnp.bfloat16),
            pltpu.VMEM((2 * ROWS, HD), jnp.bfloat16),
            pltpu.VMEM((B * NB * BS, HD), jnp.bfloat16),
            pltpu.VMEM((B * NB * BS, HD), jnp.bfloat16),
            pltpu.SemaphoreType.DMA((2,)),
            pltpu.SemaphoreType.DMA((2,)),
        ],
        compiler_params=pltpu.CompilerParams(collective_id=0),
    )(k2, v2, qmat, bt, lens)
    return out2.reshape(B, 1, H, D)


# device time: 48892 ns/iter; 1.1600x vs baseline; 1.1600x over previous
import functools

import jax
import jax.numpy as jnp
from jax import lax
from jax.experimental import pallas as pl
from jax.experimental.pallas import tpu as pltpu

B = 8
H = 8
D = 64
BS = 16
NB = 64
NPAGES_LOCAL = 64
HD = H * D
ROWS = NPAGES_LOCAL * BS
SCALE = D ** -0.5


def _body(k_ref, v_ref, qmat_ref, bt_ref, lens_ref, out_ref,
          kvf, kvs, send_sems, recv_sems):
    my_x = lax.axis_index("x")
    my_y = lax.axis_index("y")
    my_z = lax.axis_index("z")
    nbr = (1 - my_x, my_y, my_z)

    kvf[pl.ds(0, ROWS), 0:HD] = k_ref[:, :].astype(jnp.bfloat16)
    kvf[pl.ds(0, ROWS), HD:2 * HD] = v_ref[:, :].astype(jnp.bfloat16)
    kvs[:, HD:2 * HD] = jnp.zeros((ROWS, HD), jnp.bfloat16)

    barrier_sem = pltpu.get_barrier_semaphore()
    pl.semaphore_signal(barrier_sem, inc=1, device_id=nbr,
                        device_id_type=pl.DeviceIdType.MESH)
    pl.semaphore_wait(barrier_sem, 1)

    rkv = pltpu.make_async_remote_copy(
        src_ref=kvf.at[pl.ds(0, ROWS), :],
        dst_ref=kvf.at[pl.ds(ROWS, ROWS), :],
        send_sem=send_sems.at[0],
        recv_sem=recv_sems.at[0],
        device_id=nbr,
        device_id_type=pl.DeviceIdType.MESH,
    )
    rkv.start()
    rkv.wait()

    kmask = lax.broadcasted_iota(jnp.int32, (NB * BS, 1), 0)

    def per_batch(i, carry):
        def gather_one(j, c):
            p = bt_ref[i, j]
            off = (jnp.where(p // NPAGES_LOCAL == my_x, 0, ROWS)
                   + (p % NPAGES_LOCAL) * BS)
            kvs[pl.ds(j * BS, BS), :] = kvf[pl.ds(off, BS), :]
            return c
        lax.fori_loop(0, lens_ref[i], gather_one, 0)

        n_valid = lens_ref[i] * BS
        qmat = qmat_ref[pl.ds(i * HD, HD), :]
        s = lax.dot_general(
            kvs[:, 0:HD], qmat, (((1,), (0,)), ((), ())),
            preferred_element_type=jnp.float32,
        ) * SCALE
        s = jnp.where(kmask < n_valid, s, -1e30)
        m = jnp.max(s, axis=0, keepdims=True)
        p_ = jnp.exp(s - m)
        denom = jnp.sum(p_, axis=0, keepdims=True)
        r = lax.dot_general(
            p_.astype(jnp.bfloat16), kvs[:, HD:2 * HD],
            (((0,), (0,)), ((), ())),
            preferred_element_type=jnp.float32,
        )
        outs = [
            r[h:h + 1, h * D:(h + 1) * D] / denom[0:1, h:h + 1]
            for h in range(H)
        ]
        out_ref[pl.ds(i, 1), :] = jnp.concatenate(outs, axis=1)
        return carry

    lax.fori_loop(0, B, per_batch, 0)

    @functools.partial(pl.run_scoped, exit_sem=pltpu.SemaphoreType.REGULAR)
    def _(exit_sem):
        pl.semaphore_signal(exit_sem, inc=1, device_id=nbr,
                            device_id_type=pl.DeviceIdType.MESH)
        pl.semaphore_wait(exit_sem, 1)


def kernel(Q, K, V, bt, lens):
    q2 = Q.reshape(B, HD)
    k2 = K.reshape(ROWS, HD)
    v2 = V.reshape(ROWS, HD)

    head_of_col = jnp.arange(HD, dtype=jnp.int32) // D
    head_mask = head_of_col[:, None] == jnp.arange(H, dtype=jnp.int32)[None]
    qmat = (q2[:, :, None] * head_mask[None]).astype(jnp.bfloat16)
    qmat = qmat.reshape(B * HD, H)

    out2 = pl.pallas_call(
        _body,
        out_shape=jax.ShapeDtypeStruct((B, HD), jnp.float32),
        in_specs=[
            pl.BlockSpec(memory_space=pltpu.VMEM),
            pl.BlockSpec(memory_space=pltpu.VMEM),
            pl.BlockSpec(memory_space=pltpu.VMEM),
            pl.BlockSpec(memory_space=pltpu.SMEM),
            pl.BlockSpec(memory_space=pltpu.SMEM),
        ],
        out_specs=pl.BlockSpec(memory_space=pltpu.VMEM),
        scratch_shapes=[
            pltpu.VMEM((2 * ROWS, 2 * HD), jnp.bfloat16),
            pltpu.VMEM((NB * BS, 2 * HD), jnp.bfloat16),
            pltpu.SemaphoreType.DMA((1,)),
            pltpu.SemaphoreType.DMA((1,)),
        ],
        compiler_params=pltpu.CompilerParams(collective_id=0),
    )(k2, v2, qmat, bt, lens)
    return out2.reshape(B, 1, H, D)


# device time: 29508 ns/iter; 1.9221x vs baseline; 1.6569x over previous
import os

import jax
import jax.numpy as jnp
from jax import lax
from jax.experimental import pallas as pl
from jax.experimental.pallas import tpu as pltpu

B = 8
H = 8
D = 64
BS = 16
NB = 64
NPAGES_LOCAL = 64
HD = H * D
ROWS = NPAGES_LOCAL * BS
SCALE = D ** -0.5
NEG = -1e30

_NO_COMM = bool(int(os.environ.get("SCB_NO_COMM", "0")))


def _body(k_ref, v_ref, qmat_ref, bt_ref, lens_ref, out_ref,
          kvf, kvs, bias, rbuf, mdbuf, rrem, mdrem, send_sems, recv_sems):
    my_x = lax.axis_index("x")
    my_y = lax.axis_index("y")
    my_z = lax.axis_index("z")
    nbr = (1 - my_x, my_y, my_z)

    if not _NO_COMM:
        barrier_sem = pltpu.get_barrier_semaphore()
        pl.semaphore_signal(barrier_sem, inc=1, device_id=nbr,
                            device_id_type=pl.DeviceIdType.MESH)
        pl.semaphore_wait(barrier_sem, 1)

    kvf[:, 0:HD] = k_ref[:, :].astype(jnp.bfloat16)
    kvf[:, HD:2 * HD] = v_ref[:, :].astype(jnp.bfloat16)
    kvs[:, :] = jnp.zeros((ROWS, 2 * HD), jnp.bfloat16)

    for i in range(B):
        bias[:, :] = jnp.full((ROWS, 1), NEG, jnp.float32)

        def gather_one(j, c, i=i):
            p = bt_ref[i, j]

            @pl.when(p // NPAGES_LOCAL == my_x)
            def _():
                off = (p % NPAGES_LOCAL) * BS
                kvs[pl.ds(j * BS, BS), :] = kvf[pl.ds(off, BS), :]
                bias[pl.ds(j * BS, BS), :] = jnp.zeros((BS, 1), jnp.float32)
            return c
        lax.fori_loop(0, lens_ref[i], gather_one, 0)

        b = bias[:, :]
        qmat = qmat_ref[i * HD:(i + 1) * HD, :]
        s = lax.dot_general(
            kvs[:, 0:HD], qmat, (((1,), (0,)), ((), ())),
            preferred_element_type=jnp.float32,
        ) * SCALE + b
        m = jnp.max(s, axis=0, keepdims=True)
        p_ = jnp.where(b > 0.5 * NEG, jnp.exp(s - m), 0.0)
        d = jnp.sum(p_, axis=0, keepdims=True)
        r = lax.dot_general(
            p_.astype(jnp.bfloat16), kvs[:, HD:2 * HD],
            (((0,), (0,)), ((), ())),
            preferred_element_type=jnp.float32,
        )
        rbuf[i * H:(i + 1) * H, :] = r
        mdbuf[i:i + 1, :] = m
        mdbuf[B + i:B + i + 1, :] = d

    if not _NO_COMM:
        rr = pltpu.make_async_remote_copy(
            src_ref=rbuf, dst_ref=rrem,
            send_sem=send_sems.at[0], recv_sem=recv_sems.at[0],
            device_id=nbr, device_id_type=pl.DeviceIdType.MESH,
        )
        rmd = pltpu.make_async_remote_copy(
            src_ref=mdbuf, dst_ref=mdrem,
            send_sem=send_sems.at[1], recv_sem=recv_sems.at[1],
            device_id=nbr, device_id_type=pl.DeviceIdType.MESH,
        )
        rr.start()
        rmd.start()
        rr.wait()
        rmd.wait()

    for i in range(B):
        m1 = mdbuf[i:i + 1, :]
        d1 = mdbuf[B + i:B + i + 1, :]
        m2 = mdrem[i:i + 1, :]
        d2 = mdrem[B + i:B + i + 1, :]
        m = jnp.maximum(m1, m2)
        e1 = jnp.exp(m1 - m)
        e2 = jnp.exp(m2 - m)
        den = d1 * e1 + d2 * e2
        r1 = rbuf[i * H:(i + 1) * H, :]
        r2 = rrem[i * H:(i + 1) * H, :]
        outs = [
            (r1[h:h + 1, h * D:(h + 1) * D] * e1[0:1, h:h + 1]
             + r2[h:h + 1, h * D:(h + 1) * D] * e2[0:1, h:h + 1])
            / den[0:1, h:h + 1]
            for h in range(H)
        ]
        out_ref[i:i + 1, :] = jnp.concatenate(outs, axis=1)


def kernel(Q, K, V, bt, lens):
    q2 = Q.reshape(B, HD)
    k2 = K.reshape(ROWS, HD)
    v2 = V.reshape(ROWS, HD)

    head_of_col = jnp.arange(HD, dtype=jnp.int32) // D
    head_mask = head_of_col[:, None] == jnp.arange(H, dtype=jnp.int32)[None]
    qmat = (q2[:, :, None] * head_mask[None]).astype(jnp.bfloat16)
    qmat = qmat.reshape(B * HD, H)

    out2 = pl.pallas_call(
        _body,
        out_shape=jax.ShapeDtypeStruct((B, HD), jnp.float32),
        in_specs=[
            pl.BlockSpec(memory_space=pltpu.VMEM),
            pl.BlockSpec(memory_space=pltpu.VMEM),
            pl.BlockSpec(memory_space=pltpu.VMEM),
            pl.BlockSpec(memory_space=pltpu.SMEM),
            pl.BlockSpec(memory_space=pltpu.SMEM),
        ],
        out_specs=pl.BlockSpec(memory_space=pltpu.VMEM),
        scratch_shapes=[
            pltpu.VMEM((ROWS, 2 * HD), jnp.bfloat16),
            pltpu.VMEM((ROWS, 2 * HD), jnp.bfloat16),
            pltpu.VMEM((ROWS, 1), jnp.float32),
            pltpu.VMEM((B * H, HD), jnp.float32),
            pltpu.VMEM((2 * B, H), jnp.float32),
            pltpu.VMEM((B * H, HD), jnp.float32),
            pltpu.VMEM((2 * B, H), jnp.float32),
            pltpu.SemaphoreType.DMA((2,)),
            pltpu.SemaphoreType.DMA((2,)),
        ],
        compiler_params=pltpu.CompilerParams(
            collective_id=None if _NO_COMM else 0
        ),
    )(k2, v2, qmat, bt, lens)
    return out2.reshape(B, 1, H, D)


# device time: 29136 ns/iter; 1.9466x vs baseline; 1.0128x over previous
import os

import jax
import jax.numpy as jnp
from jax import lax
from jax.experimental import pallas as pl
from jax.experimental.pallas import tpu as pltpu

B = 8
H = 8
D = 64
BS = 16
NB = 64
NPAGES_LOCAL = 64
HD = H * D
ROWS = NPAGES_LOCAL * BS
SCALE = D ** -0.5
NEG = -1e30

_NO_COMM = bool(int(os.environ.get("SCB_NO_COMM", "0")))


def _body(k_ref, v_ref, qmat_ref, bt_ref, lens_ref, out_ref,
          kvf, kvs, bias, rbuf, mdbuf, rrem, mdrem, send_sems, recv_sems):
    my_x = lax.axis_index("x")
    my_y = lax.axis_index("y")
    my_z = lax.axis_index("z")
    nbr = (1 - my_x, my_y, my_z)

    if not _NO_COMM:
        barrier_sem = pltpu.get_barrier_semaphore()
        pl.semaphore_signal(barrier_sem, inc=1, device_id=nbr,
                            device_id_type=pl.DeviceIdType.MESH)
        pl.semaphore_wait(barrier_sem, 1)

    kvf[:, 0:HD] = k_ref[:, :].astype(jnp.bfloat16)
    kvf[:, HD:2 * HD] = v_ref[:, :].astype(jnp.bfloat16)
    kvs[:, :] = jnp.zeros((ROWS, 2 * HD), jnp.bfloat16)

    def r_rdma(i):
        return pltpu.make_async_remote_copy(
            src_ref=rbuf.at[pl.ds(i * H, H), :],
            dst_ref=rrem.at[pl.ds(i * H, H), :],
            send_sem=send_sems.at[i],
            recv_sem=recv_sems.at[i],
            device_id=nbr, device_id_type=pl.DeviceIdType.MESH,
        )

    def md_rdma():
        return pltpu.make_async_remote_copy(
            src_ref=mdbuf, dst_ref=mdrem,
            send_sem=send_sems.at[B],
            recv_sem=recv_sems.at[B],
            device_id=nbr, device_id_type=pl.DeviceIdType.MESH,
        )

    for i in range(B):
        bias[:, :] = jnp.full((ROWS, 1), NEG, jnp.float32)

        def gather_one(j, c, i=i):
            p = bt_ref[i, j]

            @pl.when(p // NPAGES_LOCAL == my_x)
            def _():
                off = (p % NPAGES_LOCAL) * BS
                kvs[pl.ds(j * BS, BS), :] = kvf[pl.ds(off, BS), :]
                bias[pl.ds(j * BS, BS), :] = jnp.zeros((BS, 1), jnp.float32)
            return c
        lax.fori_loop(0, lens_ref[i], gather_one, 0)

        b = bias[:, :]
        qmat = qmat_ref[i * HD:(i + 1) * HD, :]
        s = lax.dot_general(
            kvs[:, 0:HD], qmat, (((1,), (0,)), ((), ())),
            preferred_element_type=jnp.float32,
        ) * SCALE + b
        m = jnp.max(s, axis=0, keepdims=True)
        p_ = jnp.where(b > 0.5 * NEG, jnp.exp(s - m), 0.0)
        d = jnp.sum(p_, axis=0, keepdims=True)
        r = lax.dot_general(
            p_.astype(jnp.bfloat16), kvs[:, HD:2 * HD],
            (((0,), (0,)), ((), ())),
            preferred_element_type=jnp.float32,
        )
        rbuf[i * H:(i + 1) * H, :] = r
        mdbuf[i:i + 1, :] = m
        mdbuf[B + i:B + i + 1, :] = d

        if not _NO_COMM:
            r_rdma(i).start()

    if not _NO_COMM:
        md_rdma().start()
        for i in range(B):
            r_rdma(i).wait()
        md_rdma().wait()

    m1 = mdbuf[0:B, :]
    d1 = mdbuf[B:2 * B, :]
    m2 = mdrem[0:B, :]
    d2 = mdrem[B:2 * B, :]
    mm = jnp.maximum(m1, m2)
    e1 = jnp.exp(m1 - mm)
    e2 = jnp.exp(m2 - mm)
    den = d1 * e1 + d2 * e2
    e1t = jnp.transpose(e1)
    e2t = jnp.transpose(e2)
    dent = jnp.transpose(den)
    hsel = (lax.broadcasted_iota(jnp.int32, (H, HD), 1) // D
            == lax.broadcasted_iota(jnp.int32, (H, HD), 0)
            ).astype(jnp.float32)
    for i in range(B):
        r1 = rbuf[i * H:(i + 1) * H, :]
        r2 = rrem[i * H:(i + 1) * H, :]
        merged = (r1 * e1t[:, i:i + 1] + r2 * e2t[:, i:i + 1]) \
            / dent[:, i:i + 1]
        out_ref[i:i + 1, :] = jnp.sum(merged * hsel, axis=0, keepdims=True)


def kernel(Q, K, V, bt, lens):
    q2 = Q.reshape(B, HD)
    k2 = K.reshape(ROWS, HD)
    v2 = V.reshape(ROWS, HD)

    head_of_col = jnp.arange(HD, dtype=jnp.int32) // D
    head_mask = head_of_col[:, None] == jnp.arange(H, dtype=jnp.int32)[None]
    qmat = (q2[:, :, None] * head_mask[None]).astype(jnp.bfloat16)
    qmat = qmat.reshape(B * HD, H)

    out2 = pl.pallas_call(
        _body,
        out_shape=jax.ShapeDtypeStruct((B, HD), jnp.float32),
        in_specs=[
            pl.BlockSpec(memory_space=pltpu.VMEM),
            pl.BlockSpec(memory_space=pltpu.VMEM),
            pl.BlockSpec(memory_space=pltpu.VMEM),
            pl.BlockSpec(memory_space=pltpu.SMEM),
            pl.BlockSpec(memory_space=pltpu.SMEM),
        ],
        out_specs=pl.BlockSpec(memory_space=pltpu.VMEM),
        scratch_shapes=[
            pltpu.VMEM((ROWS, 2 * HD), jnp.bfloat16),
            pltpu.VMEM((ROWS, 2 * HD), jnp.bfloat16),
            pltpu.VMEM((ROWS, 1), jnp.float32),
            pltpu.VMEM((B * H, HD), jnp.float32),
            pltpu.VMEM((2 * B, H), jnp.float32),
            pltpu.VMEM((B * H, HD), jnp.float32),
            pltpu.VMEM((2 * B, H), jnp.float32),
            pltpu.SemaphoreType.DMA((B + 1,)),
            pltpu.SemaphoreType.DMA((B + 1,)),
        ],
        compiler_params=pltpu.CompilerParams(
            collective_id=None if _NO_COMM else 0
        ),
    )(k2, v2, qmat, bt, lens)
    return out2.reshape(B, 1, H, D)


# device time: 15240 ns/iter; 3.7216x vs baseline; 1.9118x over previous
import os

import jax
import jax.numpy as jnp
from jax import lax
from jax.experimental import pallas as pl
from jax.experimental.pallas import tpu as pltpu

B = 8
H = 8
D = 64
BS = 16
NB = 64
NPAGES = 128
NPAGES_LOCAL = 64
HD = H * D
BH = B * H
ROWS = NPAGES_LOCAL * BS
SCALE = D ** -0.5
NEG = -1e30

_NO_COMM = bool(int(os.environ.get("SCB_NO_COMM", "0")))


def _body(k_ref, v_ref, qm_ref, lm_ref, out_ref,
          kvf, rbuf, mdbuf, rrem, mdrem, send_sems, recv_sems):
    my_x = lax.axis_index("x")
    my_y = lax.axis_index("y")
    my_z = lax.axis_index("z")
    nbr = (1 - my_x, my_y, my_z)

    kvf[:, 0:HD] = k_ref[:, :].astype(jnp.bfloat16)
    kvf[:, HD:2 * HD] = v_ref[:, :].astype(jnp.bfloat16)

    lm = lm_ref[:, :]
    s = lax.dot_general(
        kvf[:, 0:HD], qm_ref[:, :], (((1,), (0,)), ((), ())),
        preferred_element_type=jnp.float32,
    ) * SCALE + lm
    m = jnp.max(s, axis=0, keepdims=True)
    p_ = jnp.where(lm > 0.5 * NEG, jnp.exp(s - m), 0.0)
    d = jnp.sum(p_, axis=0, keepdims=True)
    r = lax.dot_general(
        p_.astype(jnp.bfloat16), kvf[:, HD:2 * HD],
        (((0,), (0,)), ((), ())),
        preferred_element_type=jnp.float32,
    )
    rbuf[:, :] = r
    mdbuf[0:1, :] = m
    mdbuf[1:2, :] = d

    if not _NO_COMM:
        barrier_sem = pltpu.get_barrier_semaphore()
        pl.semaphore_signal(barrier_sem, inc=1, device_id=nbr,
                            device_id_type=pl.DeviceIdType.MESH)
        pl.semaphore_wait(barrier_sem, 1)

        rr = pltpu.make_async_remote_copy(
            src_ref=rbuf, dst_ref=rrem,
            send_sem=send_sems.at[0], recv_sem=recv_sems.at[0],
            device_id=nbr, device_id_type=pl.DeviceIdType.MESH,
        )
        rmd = pltpu.make_async_remote_copy(
            src_ref=mdbuf, dst_ref=mdrem,
            send_sem=send_sems.at[1], recv_sem=recv_sems.at[1],
            device_id=nbr, device_id_type=pl.DeviceIdType.MESH,
        )
        rr.start()
        rmd.start()
        rr.wait()
        rmd.wait()

    m1 = mdbuf[0:1, :]
    d1 = mdbuf[1:2, :]
    m2 = mdrem[0:1, :]
    d2 = mdrem[1:2, :]
    mm = jnp.maximum(m1, m2)
    e1 = jnp.exp(m1 - mm)
    e2 = jnp.exp(m2 - mm)
    den = d1 * e1 + d2 * e2
    e1c = jnp.transpose(e1)
    e2c = jnp.transpose(e2)
    denc = jnp.transpose(den)
    merged = (rbuf[:, :] * e1c + rrem[:, :] * e2c) / denc
    hsel = (lax.broadcasted_iota(jnp.int32, (H, HD), 1) // D
            == lax.broadcasted_iota(jnp.int32, (H, HD), 0)
            ).astype(jnp.float32)
    for i in range(B):
        mi = merged[i * H:(i + 1) * H, :]
        out_ref[i:i + 1, :] = jnp.sum(mi * hsel, axis=0, keepdims=True)


def kernel(Q, K, V, bt, lens):
    q2 = Q.reshape(B, HD)
    k2 = K.reshape(ROWS, HD)
    v2 = V.reshape(ROWS, HD)

    my_x = lax.axis_index("x")

    jmask = jnp.arange(NB, dtype=jnp.int32)[None, :] < lens[:, None]
    onehot = (bt[:, :, None] ==
              jnp.arange(NPAGES, dtype=jnp.int32)[None, None, :])
    cnt = jnp.sum(jnp.where(jmask[:, :, None], onehot, False)
                  .astype(jnp.float32), axis=1)
    cnt_my = lax.dynamic_slice(cnt, (0, my_x * NPAGES_LOCAL),
                               (B, NPAGES_LOCAL))
    logm = jnp.where(cnt_my > 0, jnp.log(cnt_my), NEG)
    lmult = jnp.repeat(jnp.repeat(logm.T, BS, axis=0), H, axis=1)

    col_head = jnp.arange(HD, dtype=jnp.int32) // D
    hm = (col_head[:, None, None] ==
          jnp.arange(H, dtype=jnp.int32)[None, None, :])
    qmatall = jnp.where(hm, q2.T[:, :, None], 0.0)
    qmatall = qmatall.astype(jnp.bfloat16).reshape(HD, BH)

    out2 = pl.pallas_call(
        _body,
        out_shape=jax.ShapeDtypeStruct((B, HD), jnp.float32),
        in_specs=[
            pl.BlockSpec(memory_space=pltpu.VMEM),
            pl.BlockSpec(memory_space=pltpu.VMEM),
            pl.BlockSpec(memory_space=pltpu.VMEM),
            pl.BlockSpec(memory_space=pltpu.VMEM),
        ],
        out_specs=pl.BlockSpec(memory_space=pltpu.VMEM),
        scratch_shapes=[
            pltpu.VMEM((ROWS, 2 * HD), jnp.bfloat16),
            pltpu.VMEM((BH, HD), jnp.float32),
            pltpu.VMEM((2, BH), jnp.float32),
            pltpu.VMEM((BH, HD), jnp.float32),
            pltpu.VMEM((2, BH), jnp.float32),
            pltpu.SemaphoreType.DMA((2,)),
            pltpu.SemaphoreType.DMA((2,)),
        ],
        compiler_params=pltpu.CompilerParams(
            collective_id=None if _NO_COMM else 0
        ),
    )(k2, v2, qmatall, lmult)
    return out2.reshape(B, 1, H, D)


# device time: 14970 ns/iter; 3.7887x vs baseline; 1.0180x over previous
import os

import jax
import jax.numpy as jnp
from jax import lax
from jax.experimental import pallas as pl
from jax.experimental.pallas import tpu as pltpu

B = 8
H = 8
D = 64
BS = 16
NB = 64
NPAGES = 128
NPAGES_LOCAL = 64
HD = H * D
BH = B * H
ROWS = NPAGES_LOCAL * BS
SCALE = D ** -0.5
NEG = -1e30

_NO_COMM = bool(int(os.environ.get("SCB_NO_COMM", "0")))


def _body(k_ref, v_ref, qt_ref, lmt_ref, out_ref,
          kvf, rbuf, mdbuf, rrem, mdrem, send_sems, recv_sems):
    my_x = lax.axis_index("x")
    my_y = lax.axis_index("y")
    my_z = lax.axis_index("z")
    nbr = (1 - my_x, my_y, my_z)

    kvf[:, 0:HD] = k_ref[:, :].astype(jnp.bfloat16)
    kvf[:, HD:2 * HD] = v_ref[:, :].astype(jnp.bfloat16)

    ecol = (lax.broadcasted_iota(jnp.int32, (B, BH), 1) // H
            == lax.broadcasted_iota(jnp.int32, (B, BH), 0)
            ).astype(jnp.float32)
    hm = (lax.broadcasted_iota(jnp.int32, (HD, BH), 0) // D
          == lax.broadcasted_iota(jnp.int32, (HD, BH), 1) % H
          ).astype(jnp.float32)
    r16 = (lax.broadcasted_iota(jnp.int32, (ROWS, NPAGES_LOCAL), 0) // BS
           == lax.broadcasted_iota(jnp.int32, (ROWS, NPAGES_LOCAL), 1)
           ).astype(jnp.float32)

    qmall = (lax.dot_general(
        qt_ref[:, :], ecol, (((1,), (0,)), ((), ())),
        preferred_element_type=jnp.float32,
    ) * hm).astype(jnp.bfloat16)
    lm8 = lax.dot_general(
        r16, lmt_ref[:, :], (((1,), (0,)), ((), ())),
        preferred_element_type=jnp.float32,
    )
    lm = lax.dot_general(
        lm8, ecol, (((1,), (0,)), ((), ())),
        preferred_element_type=jnp.float32,
    )

    s = lax.dot_general(
        kvf[:, 0:HD], qmall, (((1,), (0,)), ((), ())),
        preferred_element_type=jnp.float32,
    ) * SCALE + lm
    m = jnp.max(s, axis=0, keepdims=True)
    p_ = jnp.where(lm > 0.5 * NEG, jnp.exp(s - m), 0.0)
    d = jnp.sum(p_, axis=0, keepdims=True)
    r = lax.dot_general(
        p_.astype(jnp.bfloat16), kvf[:, HD:2 * HD],
        (((0,), (0,)), ((), ())),
        preferred_element_type=jnp.float32,
    )
    rbuf[:, :] = r
    mdbuf[0:1, :] = m
    mdbuf[1:2, :] = d

    if not _NO_COMM:
        barrier_sem = pltpu.get_barrier_semaphore()
        pl.semaphore_signal(barrier_sem, inc=1, device_id=nbr,
                            device_id_type=pl.DeviceIdType.MESH)
        pl.semaphore_wait(barrier_sem, 1)

        rr = pltpu.make_async_remote_copy(
            src_ref=rbuf, dst_ref=rrem,
            send_sem=send_sems.at[0], recv_sem=recv_sems.at[0],
            device_id=nbr, device_id_type=pl.DeviceIdType.MESH,
        )
        rmd = pltpu.make_async_remote_copy(
            src_ref=mdbuf, dst_ref=mdrem,
            send_sem=send_sems.at[1], recv_sem=recv_sems.at[1],
            device_id=nbr, device_id_type=pl.DeviceIdType.MESH,
        )
        rr.start()
        rmd.start()
        rr.wait()
        rmd.wait()

    m1 = mdbuf[0:1, :]
    d1 = mdbuf[1:2, :]
    m2 = mdrem[0:1, :]
    d2 = mdrem[1:2, :]
    mm = jnp.maximum(m1, m2)
    e1 = jnp.exp(m1 - mm)
    e2 = jnp.exp(m2 - mm)
    den = d1 * e1 + d2 * e2
    e1c = jnp.transpose(e1)
    e2c = jnp.transpose(e2)
    denc = jnp.transpose(den)
    merged = (rbuf[:, :] * e1c + rrem[:, :] * e2c) / denc
    hsel = (lax.broadcasted_iota(jnp.int32, (H, HD), 1) // D
            == lax.broadcasted_iota(jnp.int32, (H, HD), 0)
            ).astype(jnp.float32)
    for i in range(B):
        mi = merged[i * H:(i + 1) * H, :]
        out_ref[i:i + 1, :] = jnp.sum(mi * hsel, axis=0, keepdims=True)


def kernel(Q, K, V, bt, lens):
    q2 = Q.reshape(B, HD)
    k2 = K.reshape(ROWS, HD)
    v2 = V.reshape(ROWS, HD)

    my_x = lax.axis_index("x")

    jmask = jnp.arange(NB, dtype=jnp.int32)[None, :] < lens[:, None]
    onehot = (bt[:, :, None] ==
              jnp.arange(NPAGES, dtype=jnp.int32)[None, None, :])
    cnt = jnp.sum(jnp.where(jmask[:, :, None], onehot, False)
                  .astype(jnp.float32), axis=1)
    cnt_my = lax.dynamic_slice(cnt, (0, my_x * NPAGES_LOCAL),
                               (B, NPAGES_LOCAL))
    logm = jnp.where(cnt_my > 0, jnp.log(cnt_my), NEG)
    logmt = logm.T
    qt = q2.T

    out2 = pl.pallas_call(
        _body,
        out_shape=jax.ShapeDtypeStruct((B, HD), jnp.float32),
        in_specs=[
            pl.BlockSpec(memory_space=pltpu.VMEM),
            pl.BlockSpec(memory_space=pltpu.VMEM),
            pl.BlockSpec(memory_space=pltpu.VMEM),
            pl.BlockSpec(memory_space=pltpu.VMEM),
        ],
        out_specs=pl.BlockSpec(memory_space=pltpu.VMEM),
        scratch_shapes=[
            pltpu.VMEM((ROWS, 2 * HD), jnp.bfloat16),
            pltpu.VMEM((BH, HD), jnp.float32),
            pltpu.VMEM((2, BH), jnp.float32),
            pltpu.VMEM((BH, HD), jnp.float32),
            pltpu.VMEM((2, BH), jnp.float32),
            pltpu.SemaphoreType.DMA((2,)),
            pltpu.SemaphoreType.DMA((2,)),
        ],
        compiler_params=pltpu.CompilerParams(
            collective_id=None if _NO_COMM else 0
        ),
    )(k2, v2, qt, logmt)
    return out2.reshape(B, 1, H, D)
